# Initial kernel scaffold; baseline (speedup 1.0000x reference)
import jax
import jax.numpy as jnp
from jax.experimental import pallas as pl


def kernel(x, Wg, bg, W, b):
    raise NotImplementedError("write your pallas kernel here")


# speedup vs baseline: 8.8876x; 8.8876x over previous
import jax
import jax.numpy as jnp
from jax.experimental import pallas as pl
from jax.experimental.pallas import tpu as pltpu

_EMBED = 1024
_E = 16
_B = 128


def _moe_kernel(x_ref, Wg_ref, bg_ref, W_ref, b_ref, out_ref, S_acc):
    e = pl.program_id(0)

    @pl.when(e == 0)
    def _():
        S_acc[...] = jnp.zeros_like(S_acc)

    w_sum_e = jnp.sum(W_ref[0], axis=1)
    s_col = x_ref[...] @ w_sum_e[:, None]
    emask = (jax.lax.broadcasted_iota(jnp.int32, (1, _E), 1) == e).astype(
        jnp.float32)
    S_acc[...] += s_col * emask

    @pl.when(e == _E - 1)
    def _():
        logits = x_ref[...] @ Wg_ref[...] + bg_ref[...]
        m = jnp.max(logits, axis=1, keepdims=True)
        p = jnp.exp(logits - m)
        g = 1.0 / jnp.sum(p, axis=1)
        ii = jax.lax.broadcasted_iota(jnp.int32, (_B, _E), 1)
        idx = jnp.min(jnp.where(logits == m, ii, _E), axis=1)
        bsum = jnp.sum(b_ref[...], axis=1)
        A = g[:, None] * (S_acc[...] + bsum[None, :])
        H = (ii == idx[:, None]).astype(jnp.float32)
        out_ref[...] = A @ H.T


def kernel(x, Wg, bg, W, b):
    bg2 = bg.reshape(1, _E)
    return pl.pallas_call(
        _moe_kernel,
        grid=(_E,),
        in_specs=[
            pl.BlockSpec((_B, _EMBED), lambda e: (0, 0)),
            pl.BlockSpec((_EMBED, _E), lambda e: (0, 0)),
            pl.BlockSpec((1, _E), lambda e: (0, 0)),
            pl.BlockSpec((1, _EMBED, _EMBED), lambda e: (e, 0, 0)),
            pl.BlockSpec((_E, _EMBED), lambda e: (0, 0)),
        ],
        out_specs=pl.BlockSpec((_B, _B), lambda e: (0, 0)),
        out_shape=jax.ShapeDtypeStruct((_B, _B), jnp.float32),
        scratch_shapes=[
            pltpu.VMEM((_B, _E), jnp.float32),
        ],
    )(x, Wg, bg2, W, b)
